# Initial kernel scaffold; baseline (speedup 1.0000x reference)
import jax
import jax.numpy as jnp
from jax.experimental import pallas as pl


def kernel(x, edge_index, W_ext, b_ext, W1, b1, W2, b2, Wc, bc):
    raise NotImplementedError("write your pallas kernel here")


# speedup vs baseline: 19.2918x; 19.2918x over previous
import functools

import jax
import jax.numpy as jnp
from jax import lax
from jax.experimental import pallas as pl
from jax.experimental.pallas import tpu as pltpu
from jax.experimental.pallas import tpu_sc as plsc

N = 50000
NPAD = 51200
E = 800000
NW = 32
EP = 25600
EPAD = NW * EP
CH = 2560
NCHUNK = EP // CH
BN = 2048
NBLK = NPAD // BN
NR = 400
GB = 8
NG = NR // GB

_mesh = plsc.VectorSubcoreMesh(core_axis_name="c", subcore_axis_name="s")
_sc_params = pltpu.CompilerParams(needs_layout_passes=False)


def _zero_vec(ref, n):
    z16 = jnp.zeros((16,), jnp.float32)

    def body(i, carry):
        ref[pl.ds(i * 16, 16)] = z16
        return carry

    lax.fori_loop(0, n // 16, body, 0)


@functools.partial(
    pl.kernel,
    mesh=_mesh,
    compiler_params=_sc_params,
    out_type=[jax.ShapeDtypeStruct((NW, NPAD), jnp.float32)] * 2,
    scratch_types=[
        pltpu.VMEM((NPAD,), jnp.float32),
        pltpu.VMEM((NPAD,), jnp.float32),
        pltpu.VMEM((CH,), jnp.int32),
        pltpu.VMEM((CH,), jnp.int32),
    ],
)
def _sc_bincount(src_hbm, dst_hbm, outo_hbm, outi_hbm, acc_o, acc_i,
                 srcv, dstv):
    cid = lax.axis_index("c")
    sid = lax.axis_index("s")
    wid = cid * 16 + sid
    _zero_vec(acc_o, NPAD)
    _zero_vec(acc_i, NPAD)
    one16 = jnp.ones((16,), jnp.float32)

    def chunk(c, carry):
        pltpu.sync_copy(src_hbm.at[wid, pl.ds(c * CH, CH)], srcv)
        pltpu.sync_copy(dst_hbm.at[wid, pl.ds(c * CH, CH)], dstv)

        def grp(i, carry2):
            so = srcv[pl.ds(i * 16, 16)]
            do = dstv[pl.ds(i * 16, 16)]
            plsc.addupdate_scatter(acc_o, [so], one16)
            plsc.addupdate_scatter(acc_i, [do], one16)
            return carry2

        lax.fori_loop(0, CH // 16, grp, 0)
        return carry

    lax.fori_loop(0, NCHUNK, chunk, 0)
    pltpu.sync_copy(acc_o, outo_hbm.at[wid])
    pltpu.sync_copy(acc_i, outi_hbm.at[wid])


@functools.partial(
    pl.kernel,
    mesh=_mesh,
    compiler_params=_sc_params,
    out_type=jax.ShapeDtypeStruct((NW, NPAD), jnp.float32),
    scratch_types=[
        pltpu.VMEM((NPAD,), jnp.float32),
        pltpu.VMEM((NPAD,), jnp.float32),
        pltpu.VMEM((CH,), jnp.int32),
        pltpu.VMEM((CH,), jnp.int32),
    ],
)
def _sc_edge_pass(src_hbm, dst_hbm, tab_hbm, out_hbm, tabv, acc, srcv, dstv):
    cid = lax.axis_index("c")
    sid = lax.axis_index("s")
    wid = cid * 16 + sid
    pltpu.sync_copy(tab_hbm, tabv)
    _zero_vec(acc, NPAD)

    def chunk(c, carry):
        pltpu.sync_copy(src_hbm.at[wid, pl.ds(c * CH, CH)], srcv)
        pltpu.sync_copy(dst_hbm.at[wid, pl.ds(c * CH, CH)], dstv)

        def grp(i, carry2):
            so = srcv[pl.ds(i * 16, 16)]
            do = dstv[pl.ds(i * 16, 16)]
            vals = plsc.load_gather(tabv, [do])
            plsc.addupdate_scatter(acc, [so], vals)
            return carry2

        lax.fori_loop(0, CH // 16, grp, 0)
        return carry

    lax.fori_loop(0, NCHUNK, chunk, 0)
    pltpu.sync_copy(acc, out_hbm.at[wid])


def _tc_rsqrt_body(do_ref, di_ref, a_ref, dd_ref):
    do = jnp.maximum(jnp.sum(do_ref[...], axis=0), 1.0)
    di = jnp.maximum(jnp.sum(di_ref[...], axis=0), 1.0)
    dd_ref[...] = lax.rsqrt(do)
    a_ref[...] = lax.rsqrt(di)


def _tc_prep_body(w_ref, dd_ref, a_ref, p_ref, sv_ref, svacc):
    j = pl.program_id(0)

    @pl.when(j == 0)
    def _init():
        svacc[0] = 0.0

    gid = (j * GB + lax.broadcasted_iota(jnp.int32, (GB, 128), 0)) * 128 \
        + lax.broadcasted_iota(jnp.int32, (GB, 128), 1)
    m = (gid < N).astype(jnp.float32)
    v = dd_ref[...] * jnp.sum(w_ref[...], axis=0) * m
    svacc[0] += jnp.sum(v)
    p_ref[...] = v * a_ref[...]

    @pl.when(j == NG - 1)
    def _fin():
        sv_ref[...] = jnp.full((1, 1), svacc[0], jnp.float32)


def _tc_usum_body(t_ref, dd_ref, u_ref):
    j = pl.program_id(0)
    gid = (j * GB + lax.broadcasted_iota(jnp.int32, (GB, 128), 0)) * 128 \
        + lax.broadcasted_iota(jnp.int32, (GB, 128), 1)
    m = (gid < N).astype(jnp.float32)
    u_ref[...] = dd_ref[...] * jnp.sum(t_ref[...], axis=0) * m


def _tc_main_body(x_ref, u_ref, sv_ref, We, be, W1r, b1r, W2r, b2r,
                  Wcr, bcr, out_ref, zacc):
    i = pl.program_id(0)

    @pl.when(i == 0)
    def _init():
        zacc[...] = jnp.zeros_like(zacc)

    h0 = jnp.maximum(
        jnp.dot(x_ref[...], We[...], preferred_element_type=jnp.float32)
        + be[...], 0.0)
    zacc[...] += jnp.dot(u_ref[0], h0, preferred_element_type=jnp.float32)

    @pl.when(i == NBLK - 1)
    def _fin():
        z0 = zacc[...]
        sv = sv_ref[0, 0]
        z1 = jnp.dot(z0, W1r[...], preferred_element_type=jnp.float32) \
            + sv * b1r[...]
        pooled = jnp.dot(z1, W2r[...], preferred_element_type=jnp.float32) \
            * (1.0 / N) + b2r[...]
        out_ref[...] = jnp.dot(pooled, Wcr[...],
                               preferred_element_type=jnp.float32) + bcr[...]


def kernel(x, edge_index, W_ext, b_ext, W1, b1, W2, b2, Wc, bc):
    src = edge_index[0].astype(jnp.int32)
    dst = edge_index[1].astype(jnp.int32)
    padi = jnp.full((EPAD - E,), N, dtype=jnp.int32)
    srcp = jnp.concatenate([src, padi]).reshape(NW, EP)
    dstp = jnp.concatenate([dst, padi]).reshape(NW, EP)

    dego, degi = _sc_bincount(srcp, dstp)

    part_spec = pl.BlockSpec((NW, GB, 128), lambda j: (0, j, 0))
    row_spec = pl.BlockSpec((GB, 128), lambda j: (j, 0))
    a, dd = pl.pallas_call(
        _tc_rsqrt_body,
        grid=(NG,),
        in_specs=[part_spec, part_spec],
        out_specs=[row_spec, row_spec],
        out_shape=[jax.ShapeDtypeStruct((NR, 128), jnp.float32)] * 2,
    )(dego.reshape(NW, NR, 128), degi.reshape(NW, NR, 128))

    w = _sc_edge_pass(srcp, dstp, a.reshape(NPAD))

    p, sv = pl.pallas_call(
        _tc_prep_body,
        grid=(NG,),
        in_specs=[part_spec, row_spec, row_spec],
        out_specs=[row_spec, pl.BlockSpec((1, 1), lambda j: (0, 0))],
        out_shape=[jax.ShapeDtypeStruct((NR, 128), jnp.float32),
                   jax.ShapeDtypeStruct((1, 1), jnp.float32)],
        scratch_shapes=[pltpu.SMEM((1,), jnp.float32)],
    )(w.reshape(NW, NR, 128), dd, a)

    t = _sc_edge_pass(srcp, dstp, p.reshape(NPAD))

    u = pl.pallas_call(
        _tc_usum_body,
        grid=(NG,),
        in_specs=[part_spec, row_spec],
        out_specs=row_spec,
        out_shape=jax.ShapeDtypeStruct((NR, 128), jnp.float32),
    )(t.reshape(NW, NR, 128), dd)

    x_pad = jnp.concatenate(
        [x, jnp.zeros((NPAD - N, x.shape[1]), x.dtype)], axis=0)
    full = lambda shp: pl.BlockSpec(shp, lambda i: tuple(0 for _ in shp))
    out = pl.pallas_call(
        _tc_main_body,
        grid=(NBLK,),
        in_specs=[
            pl.BlockSpec((BN, 128), lambda i: (i, 0)),
            pl.BlockSpec((1, 1, BN), lambda i: (i, 0, 0)),
            full((1, 1)),
            full((128, 100)), full((1, 100)),
            full((100, 100)), full((1, 100)),
            full((100, 200)), full((1, 200)),
            full((200, 55)), full((1, 55)),
        ],
        out_specs=full((1, 55)),
        out_shape=jax.ShapeDtypeStruct((1, 55), jnp.float32),
        scratch_shapes=[pltpu.VMEM((1, 100), jnp.float32)],
    )(
        x_pad,
        u.reshape(NBLK, 1, BN),
        sv,
        W_ext, b_ext.reshape(1, 100),
        W1, b1.reshape(1, 100),
        W2, b2.reshape(1, 200),
        Wc, bc.reshape(1, 55),
    )
    return out


# speedup vs baseline: 23.1029x; 1.1976x over previous
import functools

import jax
import jax.numpy as jnp
from jax import lax
from jax.experimental import pallas as pl
from jax.experimental.pallas import tpu as pltpu
from jax.experimental.pallas import tpu_sc as plsc

N = 50000
NPAD = 51200
E = 800000
NW = 32
EP = 25600
EPAD = NW * EP
CH = 5120
NCHUNK = EP // CH
BN = 2048
NBLK = NPAD // BN
NR = 400
GB = 8
NG = NR // GB

_mesh = plsc.VectorSubcoreMesh(core_axis_name="c", subcore_axis_name="s")
_sc_params = pltpu.CompilerParams(needs_layout_passes=False)


def _edge_pipeline(src_hbm, dst_hbm, wid, bufs, sems, process_chunk):
    def start(c, b):
        sb, db = bufs[b]
        h1 = pltpu.async_copy(src_hbm.at[wid, pl.ds(c * CH, CH)], sb, sems[b])
        h2 = pltpu.async_copy(dst_hbm.at[wid, pl.ds(c * CH, CH)], db, sems[b])
        return (h1, h2)

    pend = {0: start(0, 0)}
    for c in range(NCHUNK):
        b = c % 2
        if c + 1 < NCHUNK:
            pend[(c + 1) % 2] = start(c + 1, (c + 1) % 2)
        for h in pend.pop(b):
            h.wait()
        process_chunk(*bufs[b])


@functools.partial(
    pl.kernel,
    mesh=_mesh,
    compiler_params=_sc_params,
    out_type=[jax.ShapeDtypeStruct((NW, NPAD), jnp.float32)] * 2,
    scratch_types=[
        pltpu.VMEM((NPAD,), jnp.float32),
        pltpu.VMEM((NPAD,), jnp.float32),
        pltpu.VMEM((CH,), jnp.int32),
        pltpu.VMEM((CH,), jnp.int32),
        pltpu.VMEM((CH,), jnp.int32),
        pltpu.VMEM((CH,), jnp.int32),
        pltpu.SemaphoreType.DMA,
        pltpu.SemaphoreType.DMA,
    ],
)
def _sc_bincount(src_hbm, dst_hbm, zer_hbm, outo_hbm, outi_hbm, acc_o, acc_i,
                 src0, dst0, src1, dst1, sem0, sem1):
    cid = lax.axis_index("c")
    sid = lax.axis_index("s")
    wid = cid * 16 + sid
    pltpu.sync_copy(zer_hbm, acc_o)
    pltpu.sync_copy(zer_hbm, acc_i)
    one16 = jnp.ones((16,), jnp.float32)

    def process(srcb, dstb):
        def grp(i, carry2):
            base = i * 64
            for k in range(4):
                so = srcb[pl.ds(base + k * 16, 16)]
                do = dstb[pl.ds(base + k * 16, 16)]
                plsc.addupdate_scatter(acc_o, [so], one16)
                plsc.addupdate_scatter(acc_i, [do], one16)
            return carry2

        lax.fori_loop(0, CH // 64, grp, 0)

    _edge_pipeline(src_hbm, dst_hbm, wid, [(src0, dst0), (src1, dst1)],
                   [sem0, sem1], process)
    pltpu.sync_copy(acc_o, outo_hbm.at[wid])
    pltpu.sync_copy(acc_i, outi_hbm.at[wid])


@functools.partial(
    pl.kernel,
    mesh=_mesh,
    compiler_params=_sc_params,
    out_type=jax.ShapeDtypeStruct((NW, NPAD), jnp.float32),
    scratch_types=[
        pltpu.VMEM((NPAD,), jnp.float32),
        pltpu.VMEM((NPAD,), jnp.float32),
        pltpu.VMEM((CH,), jnp.int32),
        pltpu.VMEM((CH,), jnp.int32),
        pltpu.VMEM((CH,), jnp.int32),
        pltpu.VMEM((CH,), jnp.int32),
        pltpu.SemaphoreType.DMA,
        pltpu.SemaphoreType.DMA,
        pltpu.SemaphoreType.DMA,
    ],
)
def _sc_edge_pass(src_hbm, dst_hbm, tab_hbm, zer_hbm, out_hbm, tabv, acc,
                  src0, dst0, src1, dst1, sem0, sem1, semt):
    cid = lax.axis_index("c")
    sid = lax.axis_index("s")
    wid = cid * 16 + sid
    ht = pltpu.async_copy(tab_hbm, tabv, semt)
    pltpu.sync_copy(zer_hbm, acc)
    ht.wait()

    def process(srcb, dstb):
        def grp(i, carry2):
            base = i * 64
            for k in range(4):
                so = srcb[pl.ds(base + k * 16, 16)]
                do = dstb[pl.ds(base + k * 16, 16)]
                vals = plsc.load_gather(tabv, [do])
                plsc.addupdate_scatter(acc, [so], vals)
            return carry2

        lax.fori_loop(0, CH // 64, grp, 0)

    _edge_pipeline(src_hbm, dst_hbm, wid, [(src0, dst0), (src1, dst1)],
                   [sem0, sem1], process)
    pltpu.sync_copy(acc, out_hbm.at[wid])


def _tc_rsqrt_body(do_ref, di_ref, a_ref, dd_ref):
    do = jnp.maximum(jnp.sum(do_ref[...], axis=0), 1.0)
    di = jnp.maximum(jnp.sum(di_ref[...], axis=0), 1.0)
    dd_ref[...] = lax.rsqrt(do)
    a_ref[...] = lax.rsqrt(di)


def _tc_prep_body(w_ref, dd_ref, a_ref, p_ref, sv_ref, svacc):
    j = pl.program_id(0)

    @pl.when(j == 0)
    def _init():
        svacc[0] = 0.0

    gid = (j * GB + lax.broadcasted_iota(jnp.int32, (GB, 128), 0)) * 128 \
        + lax.broadcasted_iota(jnp.int32, (GB, 128), 1)
    m = (gid < N).astype(jnp.float32)
    v = dd_ref[...] * jnp.sum(w_ref[...], axis=0) * m
    svacc[0] += jnp.sum(v)
    p_ref[...] = v * a_ref[...]

    @pl.when(j == NG - 1)
    def _fin():
        sv_ref[...] = jnp.full((1, 1), svacc[0], jnp.float32)


def _tc_usum_body(t_ref, dd_ref, u_ref):
    j = pl.program_id(0)
    gid = (j * GB + lax.broadcasted_iota(jnp.int32, (GB, 128), 0)) * 128 \
        + lax.broadcasted_iota(jnp.int32, (GB, 128), 1)
    m = (gid < N).astype(jnp.float32)
    u_ref[...] = dd_ref[...] * jnp.sum(t_ref[...], axis=0) * m


def _tc_main_body(x_ref, u_ref, sv_ref, We, be, W1r, b1r, W2r, b2r,
                  Wcr, bcr, out_ref, zacc):
    i = pl.program_id(0)

    @pl.when(i == 0)
    def _init():
        zacc[...] = jnp.zeros_like(zacc)

    h0 = jnp.maximum(
        jnp.dot(x_ref[...], We[...], preferred_element_type=jnp.float32)
        + be[...], 0.0)
    zacc[...] += jnp.dot(u_ref[0], h0, preferred_element_type=jnp.float32)

    @pl.when(i == NBLK - 1)
    def _fin():
        z0 = zacc[...]
        sv = sv_ref[0, 0]
        z1 = jnp.dot(z0, W1r[...], preferred_element_type=jnp.float32) \
            + sv * b1r[...]
        pooled = jnp.dot(z1, W2r[...], preferred_element_type=jnp.float32) \
            * (1.0 / N) + b2r[...]
        out_ref[...] = jnp.dot(pooled, Wcr[...],
                               preferred_element_type=jnp.float32) + bcr[...]


def kernel(x, edge_index, W_ext, b_ext, W1, b1, W2, b2, Wc, bc):
    src = edge_index[0].astype(jnp.int32)
    dst = edge_index[1].astype(jnp.int32)
    padi = jnp.full((EPAD - E,), N, dtype=jnp.int32)
    srcp = jnp.concatenate([src, padi]).reshape(NW, EP)
    dstp = jnp.concatenate([dst, padi]).reshape(NW, EP)

    zer = jnp.zeros((NPAD,), jnp.float32)
    dego, degi = _sc_bincount(srcp, dstp, zer)

    part_spec = pl.BlockSpec((NW, GB, 128), lambda j: (0, j, 0))
    row_spec = pl.BlockSpec((GB, 128), lambda j: (j, 0))
    a, dd = pl.pallas_call(
        _tc_rsqrt_body,
        grid=(NG,),
        in_specs=[part_spec, part_spec],
        out_specs=[row_spec, row_spec],
        out_shape=[jax.ShapeDtypeStruct((NR, 128), jnp.float32)] * 2,
    )(dego.reshape(NW, NR, 128), degi.reshape(NW, NR, 128))

    w = _sc_edge_pass(srcp, dstp, a.reshape(NPAD), zer)

    p, sv = pl.pallas_call(
        _tc_prep_body,
        grid=(NG,),
        in_specs=[part_spec, row_spec, row_spec],
        out_specs=[row_spec, pl.BlockSpec((1, 1), lambda j: (0, 0))],
        out_shape=[jax.ShapeDtypeStruct((NR, 128), jnp.float32),
                   jax.ShapeDtypeStruct((1, 1), jnp.float32)],
        scratch_shapes=[pltpu.SMEM((1,), jnp.float32)],
    )(w.reshape(NW, NR, 128), dd, a)

    t = _sc_edge_pass(srcp, dstp, p.reshape(NPAD), zer)

    u = pl.pallas_call(
        _tc_usum_body,
        grid=(NG,),
        in_specs=[part_spec, row_spec],
        out_specs=row_spec,
        out_shape=jax.ShapeDtypeStruct((NR, 128), jnp.float32),
    )(t.reshape(NW, NR, 128), dd)

    x_pad = jnp.concatenate(
        [x, jnp.zeros((NPAD - N, x.shape[1]), x.dtype)], axis=0)
    full = lambda shp: pl.BlockSpec(shp, lambda i: tuple(0 for _ in shp))
    out = pl.pallas_call(
        _tc_main_body,
        grid=(NBLK,),
        in_specs=[
            pl.BlockSpec((BN, 128), lambda i: (i, 0)),
            pl.BlockSpec((1, 1, BN), lambda i: (i, 0, 0)),
            full((1, 1)),
            full((128, 100)), full((1, 100)),
            full((100, 100)), full((1, 100)),
            full((100, 200)), full((1, 200)),
            full((200, 55)), full((1, 55)),
        ],
        out_specs=full((1, 55)),
        out_shape=jax.ShapeDtypeStruct((1, 55), jnp.float32),
        scratch_shapes=[pltpu.VMEM((1, 100), jnp.float32)],
    )(
        x_pad,
        u.reshape(NBLK, 1, BN),
        sv,
        W_ext, b_ext.reshape(1, 100),
        W1, b1.reshape(1, 100),
        W2, b2.reshape(1, 200),
        Wc, bc.reshape(1, 55),
    )
    return out


# speedup vs baseline: 25.7736x; 1.1156x over previous
import functools

import jax
import jax.numpy as jnp
from jax import lax
from jax.experimental import pallas as pl
from jax.experimental.pallas import tpu as pltpu
from jax.experimental.pallas import tpu_sc as plsc

N = 50000
NPAD = 51200
E = 800000
NW = 32
EP = 25600
EPAD = NW * EP
CH = 5120
CB = CH // 128
NCHUNK = EP // CH
SL = NPAD // 16
BN = 2048
NBLK = NPAD // BN
NR = 400
GB = 8
NG = NR // GB

_mesh = plsc.VectorSubcoreMesh(core_axis_name="c", subcore_axis_name="s")
_sc_params = pltpu.CompilerParams(needs_layout_passes=False)


def _edge_pipeline(src_hbm, dst_hbm, wid, bufs, sems, process_chunk):
    def start(c, b):
        sb, db = bufs[b]
        h1 = pltpu.async_copy(
            src_hbm.at[wid, pl.ds(c * CB, CB), slice(None)], sb, sems[b])
        h2 = pltpu.async_copy(
            dst_hbm.at[wid, pl.ds(c * CB, CB), slice(None)], db, sems[b])
        return (h1, h2)

    pend = {0: start(0, 0)}
    for c in range(NCHUNK):
        b = c % 2
        if c + 1 < NCHUNK:
            pend[(c + 1) % 2] = start(c + 1, (c + 1) % 2)
        for h in pend.pop(b):
            h.wait()
        process_chunk(*bufs[b])


@functools.partial(
    pl.kernel,
    mesh=_mesh,
    compiler_params=_sc_params,
    out_type=[jax.ShapeDtypeStruct((2, NPAD), jnp.float32)] * 2,
    scratch_types=[
        pltpu.VMEM((128,), jnp.float32),
        pltpu.VMEM((SL,), jnp.float32),
        pltpu.VMEM((CB, 128), jnp.int32),
        pltpu.VMEM((CB, 128), jnp.int32),
        pltpu.VMEM((CB, 128), jnp.int32),
        pltpu.VMEM((CB, 128), jnp.int32),
        pltpu.VMEM_SHARED((NPAD,), jnp.float32),
        pltpu.VMEM_SHARED((NPAD,), jnp.float32),
        pltpu.SemaphoreType.DMA,
        pltpu.SemaphoreType.DMA,
    ],
)
def _sc_bincount(src_hbm, dst_hbm, outo_hbm, outi_hbm, ones, zbuf,
                 src0, dst0, src1, dst1, acc_o, acc_i, sem0, sem1):
    cid = lax.axis_index("c")
    sid = lax.axis_index("s")
    wid = cid * 16 + sid
    one16 = jnp.ones((16,), jnp.float32)
    z16 = jnp.zeros((16,), jnp.float32)

    for i in range(8):
        ones[pl.ds(i * 16, 16)] = one16

    def fill(i, carry):
        zbuf[pl.ds(i * 16, 16)] = z16
        return carry

    lax.fori_loop(0, SL // 16, fill, 0)
    pltpu.sync_copy(zbuf, acc_o.at[pl.ds(sid * SL, SL)])
    pltpu.sync_copy(zbuf, acc_i.at[pl.ds(sid * SL, SL)])
    plsc.subcore_barrier()

    def process(srcb, dstb):
        def row(b, carry2):
            pltpu.sync_copy(ones, acc_o.at[srcb.at[b]], add=True)
            pltpu.sync_copy(ones, acc_i.at[dstb.at[b]], add=True)
            return carry2

        lax.fori_loop(0, CB, row, 0)

    _edge_pipeline(src_hbm, dst_hbm, wid, [(src0, dst0), (src1, dst1)],
                   [sem0, sem1], process)
    plsc.subcore_barrier()

    @pl.when(sid == 0)
    def _out():
        pltpu.sync_copy(acc_o, outo_hbm.at[cid])
        pltpu.sync_copy(acc_i, outi_hbm.at[cid])


@functools.partial(
    pl.kernel,
    mesh=_mesh,
    compiler_params=_sc_params,
    out_type=jax.ShapeDtypeStruct((2, NPAD), jnp.float32),
    scratch_types=[
        pltpu.VMEM((NPAD,), jnp.float32),
        pltpu.VMEM((CB, 128), jnp.float32),
        pltpu.VMEM((SL,), jnp.float32),
        pltpu.VMEM((CB, 128), jnp.int32),
        pltpu.VMEM((CB, 128), jnp.int32),
        pltpu.VMEM((CB, 128), jnp.int32),
        pltpu.VMEM((CB, 128), jnp.int32),
        pltpu.VMEM_SHARED((NPAD,), jnp.float32),
        pltpu.SemaphoreType.DMA,
        pltpu.SemaphoreType.DMA,
        pltpu.SemaphoreType.DMA,
    ],
)
def _sc_edge_pass(src_hbm, dst_hbm, tab_hbm, out_hbm, tabv, vals, zbuf,
                  src0, dst0, src1, dst1, acc, sem0, sem1, semt):
    cid = lax.axis_index("c")
    sid = lax.axis_index("s")
    wid = cid * 16 + sid
    ht = pltpu.async_copy(tab_hbm, tabv, semt)
    z16 = jnp.zeros((16,), jnp.float32)

    def fill(i, carry):
        zbuf[pl.ds(i * 16, 16)] = z16
        return carry

    lax.fori_loop(0, SL // 16, fill, 0)
    pltpu.sync_copy(zbuf, acc.at[pl.ds(sid * SL, SL)])
    ht.wait()
    plsc.subcore_barrier()

    def process(srcb, dstb):
        def grp(i, carry2):
            r = i // 2
            half = (i % 2) * 64
            for k in range(4):
                off = half + k * 16
                do = dstb[r, pl.ds(off, 16)]
                vals[r, pl.ds(off, 16)] = plsc.load_gather(tabv, [do])
            return carry2

        lax.fori_loop(0, CB * 2, grp, 0)

        def row(b, carry2):
            pltpu.sync_copy(vals.at[b], acc.at[srcb.at[b]], add=True)
            return carry2

        lax.fori_loop(0, CB, row, 0)

    _edge_pipeline(src_hbm, dst_hbm, wid, [(src0, dst0), (src1, dst1)],
                   [sem0, sem1], process)
    plsc.subcore_barrier()

    @pl.when(sid == 0)
    def _out():
        pltpu.sync_copy(acc, out_hbm.at[cid])


def _tc_rsqrt_body(do_ref, di_ref, a_ref, dd_ref):
    do = jnp.maximum(jnp.sum(do_ref[...], axis=0), 1.0)
    di = jnp.maximum(jnp.sum(di_ref[...], axis=0), 1.0)
    dd_ref[...] = lax.rsqrt(do)
    a_ref[...] = lax.rsqrt(di)


def _tc_prep_body(w_ref, dd_ref, a_ref, p_ref, sv_ref, svacc):
    j = pl.program_id(0)

    @pl.when(j == 0)
    def _init():
        svacc[0] = 0.0

    gid = (j * GB + lax.broadcasted_iota(jnp.int32, (GB, 128), 0)) * 128 \
        + lax.broadcasted_iota(jnp.int32, (GB, 128), 1)
    m = (gid < N).astype(jnp.float32)
    v = dd_ref[...] * jnp.sum(w_ref[...], axis=0) * m
    svacc[0] += jnp.sum(v)
    p_ref[...] = v * a_ref[...]

    @pl.when(j == NG - 1)
    def _fin():
        sv_ref[...] = jnp.full((1, 1), svacc[0], jnp.float32)


def _tc_usum_body(t_ref, dd_ref, u_ref):
    j = pl.program_id(0)
    gid = (j * GB + lax.broadcasted_iota(jnp.int32, (GB, 128), 0)) * 128 \
        + lax.broadcasted_iota(jnp.int32, (GB, 128), 1)
    m = (gid < N).astype(jnp.float32)
    u_ref[...] = dd_ref[...] * jnp.sum(t_ref[...], axis=0) * m


def _tc_main_body(x_ref, u_ref, sv_ref, We, be, W1r, b1r, W2r, b2r,
                  Wcr, bcr, out_ref, zacc):
    i = pl.program_id(0)

    @pl.when(i == 0)
    def _init():
        zacc[...] = jnp.zeros_like(zacc)

    h0 = jnp.maximum(
        jnp.dot(x_ref[...], We[...], preferred_element_type=jnp.float32,
                precision=lax.Precision.HIGHEST)
        + be[...], 0.0)
    zacc[...] += jnp.dot(u_ref[0], h0, preferred_element_type=jnp.float32,
                         precision=lax.Precision.HIGHEST)

    @pl.when(i == NBLK - 1)
    def _fin():
        z0 = zacc[...]
        sv = sv_ref[0, 0]
        z1 = jnp.dot(z0, W1r[...], preferred_element_type=jnp.float32) \
            + sv * b1r[...]
        pooled = jnp.dot(z1, W2r[...], preferred_element_type=jnp.float32) \
            * (1.0 / N) + b2r[...]
        out_ref[...] = jnp.dot(pooled, Wcr[...],
                               preferred_element_type=jnp.float32) + bcr[...]


def kernel(x, edge_index, W_ext, b_ext, W1, b1, W2, b2, Wc, bc):
    src = edge_index[0].astype(jnp.int32)
    dst = edge_index[1].astype(jnp.int32)
    padi = jnp.full((EPAD - E,), N, dtype=jnp.int32)
    srcp = jnp.concatenate([src, padi]).reshape(NW, EP // 128, 128)
    dstp = jnp.concatenate([dst, padi]).reshape(NW, EP // 128, 128)

    dego, degi = _sc_bincount(srcp, dstp)

    part_spec = pl.BlockSpec((2, GB, 128), lambda j: (0, j, 0))
    row_spec = pl.BlockSpec((GB, 128), lambda j: (j, 0))
    a, dd = pl.pallas_call(
        _tc_rsqrt_body,
        grid=(NG,),
        in_specs=[part_spec, part_spec],
        out_specs=[row_spec, row_spec],
        out_shape=[jax.ShapeDtypeStruct((NR, 128), jnp.float32)] * 2,
    )(dego.reshape(2, NR, 128), degi.reshape(2, NR, 128))

    w = _sc_edge_pass(srcp, dstp, a.reshape(NPAD))

    p, sv = pl.pallas_call(
        _tc_prep_body,
        grid=(NG,),
        in_specs=[part_spec, row_spec, row_spec],
        out_specs=[row_spec, pl.BlockSpec((1, 1), lambda j: (0, 0))],
        out_shape=[jax.ShapeDtypeStruct((NR, 128), jnp.float32),
                   jax.ShapeDtypeStruct((1, 1), jnp.float32)],
        scratch_shapes=[pltpu.SMEM((1,), jnp.float32)],
    )(w.reshape(2, NR, 128), dd, a)

    t = _sc_edge_pass(srcp, dstp, p.reshape(NPAD))

    u = pl.pallas_call(
        _tc_usum_body,
        grid=(NG,),
        in_specs=[part_spec, row_spec],
        out_specs=row_spec,
        out_shape=jax.ShapeDtypeStruct((NR, 128), jnp.float32),
    )(t.reshape(2, NR, 128), dd)

    x_pad = jnp.concatenate(
        [x, jnp.zeros((NPAD - N, x.shape[1]), x.dtype)], axis=0)
    full = lambda shp: pl.BlockSpec(shp, lambda i: tuple(0 for _ in shp))
    out = pl.pallas_call(
        _tc_main_body,
        grid=(NBLK,),
        in_specs=[
            pl.BlockSpec((BN, 128), lambda i: (i, 0)),
            pl.BlockSpec((1, 1, BN), lambda i: (i, 0, 0)),
            full((1, 1)),
            full((128, 100)), full((1, 100)),
            full((100, 100)), full((1, 100)),
            full((100, 200)), full((1, 200)),
            full((200, 55)), full((1, 55)),
        ],
        out_specs=full((1, 55)),
        out_shape=jax.ShapeDtypeStruct((1, 55), jnp.float32),
        scratch_shapes=[pltpu.VMEM((1, 100), jnp.float32)],
    )(
        x_pad,
        u.reshape(NBLK, 1, BN),
        sv,
        W_ext, b_ext.reshape(1, 100),
        W1, b1.reshape(1, 100),
        W2, b2.reshape(1, 200),
        Wc, bc.reshape(1, 55),
    )
    return out


# speedup vs baseline: 31.8193x; 1.2346x over previous
import functools

import jax
import jax.numpy as jnp
from jax import lax
from jax.experimental import pallas as pl
from jax.experimental.pallas import tpu as pltpu
from jax.experimental.pallas import tpu_sc as plsc

N = 50000
NPAD = 51200
E = 800000
NW = 32
EP = 25600
EPAD = NW * EP
CH = 5120
CB = CH // 128
NCHUNK = EP // CH
SL = NPAD // 16
BN = 2048
NBLK = NPAD // BN
NR = 400
GB = 8
NG = NR // GB

_mesh = plsc.VectorSubcoreMesh(core_axis_name="c", subcore_axis_name="s")
_sc_params = pltpu.CompilerParams(needs_layout_passes=False)


def _edge_pipeline(src_hbm, dst_hbm, wid, bufs, sems, process_chunk):
    def start(c, b):
        sb, db = bufs[b]
        h1 = pltpu.async_copy(
            src_hbm.at[wid, pl.ds(c * CB, CB), slice(None)], sb, sems[b])
        h2 = pltpu.async_copy(
            dst_hbm.at[wid, pl.ds(c * CB, CB), slice(None)], db, sems[b])
        return (h1, h2)

    pend = {0: start(0, 0)}
    for c in range(NCHUNK):
        b = c % 2
        if c + 1 < NCHUNK:
            pend[(c + 1) % 2] = start(c + 1, (c + 1) % 2)
        for h in pend.pop(b):
            h.wait()
        process_chunk(*bufs[b])


def _rsqrt16(x):
    xi = plsc.bitcast(x, jnp.int32)
    yi = jnp.full((16,), 0x5F3759DF, jnp.int32) - lax.shift_right_arithmetic(
        xi, jnp.full((16,), 1, jnp.int32))
    y = plsc.bitcast(yi, jnp.float32)
    for _ in range(4):
        y = y * (1.5 - 0.5 * x * y * y)
    return y


@functools.partial(
    pl.kernel,
    mesh=_mesh,
    compiler_params=_sc_params,
    out_type=[jax.ShapeDtypeStruct((2, NPAD), jnp.float32)] * 2,
    scratch_types=[
        pltpu.VMEM((128,), jnp.float32),
        pltpu.VMEM((SL,), jnp.float32),
        pltpu.VMEM((CB, 128), jnp.int32),
        pltpu.VMEM((CB, 128), jnp.int32),
        pltpu.VMEM((CB, 128), jnp.int32),
        pltpu.VMEM((CB, 128), jnp.int32),
        pltpu.VMEM_SHARED((NPAD,), jnp.float32),
        pltpu.VMEM_SHARED((NPAD,), jnp.float32),
        pltpu.SemaphoreType.DMA,
        pltpu.SemaphoreType.DMA,
    ],
)
def _sc_bincount(src_hbm, dst_hbm, outo_hbm, outi_hbm, ones, zbuf,
                 src0, dst0, src1, dst1, acc_o, acc_i, sem0, sem1):
    cid = lax.axis_index("c")
    sid = lax.axis_index("s")
    wid = cid * 16 + sid
    one16 = jnp.ones((16,), jnp.float32)
    z16 = jnp.zeros((16,), jnp.float32)

    for i in range(8):
        ones[pl.ds(i * 16, 16)] = one16

    def fill(i, carry):
        zbuf[pl.ds(i * 16, 16)] = z16
        return carry

    lax.fori_loop(0, SL // 16, fill, 0)
    pltpu.sync_copy(zbuf, acc_o.at[pl.ds(sid * SL, SL)])
    pltpu.sync_copy(zbuf, acc_i.at[pl.ds(sid * SL, SL)])
    plsc.subcore_barrier()

    def process(srcb, dstb):
        def row(b, carry2):
            pltpu.sync_copy(ones, acc_o.at[srcb.at[b]], add=True)
            pltpu.sync_copy(ones, acc_i.at[dstb.at[b]], add=True)
            return carry2

        lax.fori_loop(0, CB, row, 0)

    _edge_pipeline(src_hbm, dst_hbm, wid, [(src0, dst0), (src1, dst1)],
                   [sem0, sem1], process)
    plsc.subcore_barrier()

    @pl.when(sid == 0)
    def _out():
        pltpu.sync_copy(acc_o, outo_hbm.at[cid])
        pltpu.sync_copy(acc_i, outi_hbm.at[cid])


@functools.partial(
    pl.kernel,
    mesh=_mesh,
    compiler_params=_sc_params,
    out_type=jax.ShapeDtypeStruct((2, NPAD), jnp.float32),
    scratch_types=[
        pltpu.VMEM((NPAD,), jnp.float32),
        pltpu.VMEM((CB, 128), jnp.float32),
        pltpu.VMEM((SL,), jnp.float32),
        pltpu.VMEM((SL,), jnp.float32),
        pltpu.VMEM((CB, 128), jnp.int32),
        pltpu.VMEM((CB, 128), jnp.int32),
        pltpu.VMEM((CB, 128), jnp.int32),
        pltpu.VMEM((CB, 128), jnp.int32),
        pltpu.VMEM_SHARED((NPAD,), jnp.float32),
        pltpu.VMEM_SHARED((NPAD,), jnp.float32),
        pltpu.SemaphoreType.DMA,
        pltpu.SemaphoreType.DMA,
    ],
)
def _sc_pass1(src_hbm, dst_hbm, degi_hbm, out_hbm, tabv, vals, sb0, sb1,
              src0, dst0, src1, dst1, tab_sh, acc, sem0, sem1):
    cid = lax.axis_index("c")
    sid = lax.axis_index("s")
    wid = cid * 16 + sid
    base = sid * SL
    pltpu.sync_copy(degi_hbm.at[0, pl.ds(base, SL)], sb0)
    pltpu.sync_copy(degi_hbm.at[1, pl.ds(base, SL)], sb1)
    z16 = jnp.zeros((16,), jnp.float32)

    def prol(i, carry):
        d = jnp.maximum(sb0[pl.ds(i * 16, 16)] + sb1[pl.ds(i * 16, 16)], 1.0)
        sb0[pl.ds(i * 16, 16)] = _rsqrt16(d)
        sb1[pl.ds(i * 16, 16)] = z16
        return carry

    lax.fori_loop(0, SL // 16, prol, 0)
    pltpu.sync_copy(sb0, tab_sh.at[pl.ds(base, SL)])
    pltpu.sync_copy(sb1, acc.at[pl.ds(base, SL)])
    plsc.subcore_barrier()
    pltpu.sync_copy(tab_sh, tabv)

    def process(srcb, dstb):
        def grp(i, carry2):
            r = i // 2
            half = (i % 2) * 64
            for k in range(4):
                off = half + k * 16
                do = dstb[r, pl.ds(off, 16)]
                vals[r, pl.ds(off, 16)] = plsc.load_gather(tabv, [do])
            return carry2

        lax.fori_loop(0, CB * 2, grp, 0)

        def row(b, carry2):
            pltpu.sync_copy(vals.at[b], acc.at[srcb.at[b]], add=True)
            return carry2

        lax.fori_loop(0, CB, row, 0)

    _edge_pipeline(src_hbm, dst_hbm, wid, [(src0, dst0), (src1, dst1)],
                   [sem0, sem1], process)
    plsc.subcore_barrier()

    @pl.when(sid == 0)
    def _out():
        pltpu.sync_copy(acc, out_hbm.at[cid])


@functools.partial(
    pl.kernel,
    mesh=_mesh,
    compiler_params=_sc_params,
    out_type=[jax.ShapeDtypeStruct((2, NPAD), jnp.float32),
              jax.ShapeDtypeStruct((NPAD,), jnp.float32),
              jax.ShapeDtypeStruct((NPAD,), jnp.float32)],
    scratch_types=[
        pltpu.VMEM((NPAD,), jnp.float32),
        pltpu.VMEM((CB, 128), jnp.float32),
        pltpu.VMEM((SL,), jnp.float32),
        pltpu.VMEM((SL,), jnp.float32),
        pltpu.VMEM((SL,), jnp.float32),
        pltpu.VMEM((SL,), jnp.float32),
        pltpu.VMEM((SL,), jnp.float32),
        pltpu.VMEM((SL,), jnp.float32),
        pltpu.VMEM((CB, 128), jnp.int32),
        pltpu.VMEM((CB, 128), jnp.int32),
        pltpu.VMEM((CB, 128), jnp.int32),
        pltpu.VMEM((CB, 128), jnp.int32),
        pltpu.VMEM_SHARED((NPAD,), jnp.float32),
        pltpu.VMEM_SHARED((NPAD,), jnp.float32),
        pltpu.SemaphoreType.DMA,
        pltpu.SemaphoreType.DMA,
    ],
)
def _sc_pass2(src_hbm, dst_hbm, dego_hbm, degi_hbm, w_hbm, out_hbm, dd_hbm,
              v_hbm, tabv, vals, bi0, bi1, bo0, bo1, bw0, bw1,
              src0, dst0, src1, dst1, tab_sh, acc, sem0, sem1):
    cid = lax.axis_index("c")
    sid = lax.axis_index("s")
    wid = cid * 16 + sid
    base = sid * SL
    pltpu.sync_copy(degi_hbm.at[0, pl.ds(base, SL)], bi0)
    pltpu.sync_copy(degi_hbm.at[1, pl.ds(base, SL)], bi1)
    pltpu.sync_copy(dego_hbm.at[0, pl.ds(base, SL)], bo0)
    pltpu.sync_copy(dego_hbm.at[1, pl.ds(base, SL)], bo1)
    pltpu.sync_copy(w_hbm.at[0, pl.ds(base, SL)], bw0)
    pltpu.sync_copy(w_hbm.at[1, pl.ds(base, SL)], bw1)
    z16 = jnp.zeros((16,), jnp.float32)

    def prol(i, carry):
        sl = pl.ds(i * 16, 16)
        a = _rsqrt16(jnp.maximum(bi0[sl] + bi1[sl], 1.0))
        d = _rsqrt16(jnp.maximum(bo0[sl] + bo1[sl], 1.0))
        v = d * (bw0[sl] + bw1[sl])
        bo0[sl] = d
        bw0[sl] = v
        bi0[sl] = v * a
        bi1[sl] = z16
        return carry

    lax.fori_loop(0, SL // 16, prol, 0)

    @pl.when(cid == 0)
    def _ddv_out():
        pltpu.sync_copy(bo0, dd_hbm.at[pl.ds(base, SL)])
        pltpu.sync_copy(bw0, v_hbm.at[pl.ds(base, SL)])

    pltpu.sync_copy(bi0, tab_sh.at[pl.ds(base, SL)])
    pltpu.sync_copy(bi1, acc.at[pl.ds(base, SL)])
    plsc.subcore_barrier()
    pltpu.sync_copy(tab_sh, tabv)

    def process(srcb, dstb):
        def grp(i, carry2):
            r = i // 2
            half = (i % 2) * 64
            for k in range(4):
                off = half + k * 16
                do = dstb[r, pl.ds(off, 16)]
                vals[r, pl.ds(off, 16)] = plsc.load_gather(tabv, [do])
            return carry2

        lax.fori_loop(0, CB * 2, grp, 0)

        def row(b, carry2):
            pltpu.sync_copy(vals.at[b], acc.at[srcb.at[b]], add=True)
            return carry2

        lax.fori_loop(0, CB, row, 0)

    _edge_pipeline(src_hbm, dst_hbm, wid, [(src0, dst0), (src1, dst1)],
                   [sem0, sem1], process)
    plsc.subcore_barrier()

    @pl.when(sid == 0)
    def _out():
        pltpu.sync_copy(acc, out_hbm.at[cid])


def _tc_main_body(x_ref, t0, t1, ddr, vr, We, be, W1r, b1r, W2r, b2r,
                  Wcr, bcr, out_ref, zacc, svacc):
    i = pl.program_id(0)

    @pl.when(i == 0)
    def _init():
        zacc[...] = jnp.zeros_like(zacc)
        svacc[0] = 0.0

    idx = i * BN + lax.broadcasted_iota(jnp.int32, (1, BN), 1)
    m = (idx < N).astype(jnp.float32)
    u = ddr[0] * (t0[0] + t1[0]) * m
    svacc[0] += jnp.sum(vr[0] * m)
    h0 = jnp.maximum(
        jnp.dot(x_ref[...], We[...], preferred_element_type=jnp.float32,
                precision=lax.Precision.HIGHEST)
        + be[...], 0.0)
    zacc[...] += jnp.dot(u, h0, preferred_element_type=jnp.float32,
                         precision=lax.Precision.HIGHEST)

    @pl.when(i == NBLK - 1)
    def _fin():
        z0 = zacc[...]
        sv = svacc[0]
        z1 = jnp.dot(z0, W1r[...], preferred_element_type=jnp.float32) \
            + sv * b1r[...]
        pooled = jnp.dot(z1, W2r[...], preferred_element_type=jnp.float32) \
            * (1.0 / N) + b2r[...]
        out_ref[...] = jnp.dot(pooled, Wcr[...],
                               preferred_element_type=jnp.float32) + bcr[...]


def kernel(x, edge_index, W_ext, b_ext, W1, b1, W2, b2, Wc, bc):
    src = edge_index[0].astype(jnp.int32)
    dst = edge_index[1].astype(jnp.int32)
    padi = jnp.full((EPAD - E,), N, dtype=jnp.int32)
    srcp = jnp.concatenate([src, padi]).reshape(NW, EP // 128, 128)
    dstp = jnp.concatenate([dst, padi]).reshape(NW, EP // 128, 128)

    dego, degi = _sc_bincount(srcp, dstp)
    w = _sc_pass1(srcp, dstp, degi)
    t, dd, v = _sc_pass2(srcp, dstp, dego, degi, w)

    x_pad = jnp.concatenate(
        [x, jnp.zeros((NPAD - N, x.shape[1]), x.dtype)], axis=0)
    full = lambda shp: pl.BlockSpec(shp, lambda i: tuple(0 for _ in shp))
    row = pl.BlockSpec((1, 1, BN), lambda i: (i, 0, 0))
    out = pl.pallas_call(
        _tc_main_body,
        grid=(NBLK,),
        in_specs=[
            pl.BlockSpec((BN, 128), lambda i: (i, 0)),
            row, row,
            row, row,
            full((128, 100)), full((1, 100)),
            full((100, 100)), full((1, 100)),
            full((100, 200)), full((1, 200)),
            full((200, 55)), full((1, 55)),
        ],
        out_specs=full((1, 55)),
        out_shape=jax.ShapeDtypeStruct((1, 55), jnp.float32),
        scratch_shapes=[pltpu.VMEM((1, 100), jnp.float32),
                        pltpu.SMEM((1,), jnp.float32)],
    )(
        x_pad,
        t[0].reshape(NBLK, 1, BN), t[1].reshape(NBLK, 1, BN),
        dd.reshape(NBLK, 1, BN), v.reshape(NBLK, 1, BN),
        W_ext, b_ext.reshape(1, 100),
        W1, b1.reshape(1, 100),
        W2, b2.reshape(1, 200),
        Wc, bc.reshape(1, 55),
    )
    return out


# speedup vs baseline: 36.0127x; 1.1318x over previous
import functools

import jax
import jax.numpy as jnp
from jax import lax
from jax.experimental import pallas as pl
from jax.experimental.pallas import tpu as pltpu
from jax.experimental.pallas import tpu_sc as plsc

N = 50000
NPAD = 51200
E = 800000
NW = 32
EP = 25600
EPAD = NW * EP
CH = 5120
CB = CH // 128
NCHUNK = EP // CH
SL = NPAD // 16
BN = 2048
NBLK = NPAD // BN
NR = 400
GB = 8
NG = NR // GB

_mesh = plsc.VectorSubcoreMesh(core_axis_name="c", subcore_axis_name="s")
_sc_params = pltpu.CompilerParams(needs_layout_passes=False)


def _edge_pipeline(src_hbm, dst_hbm, wid, bufs, sems, process_chunk):
    def start(c, b):
        sb, db = bufs[b]
        h1 = pltpu.async_copy(
            src_hbm.at[wid, pl.ds(c * CB, CB), slice(None)], sb, sems[b])
        h2 = pltpu.async_copy(
            dst_hbm.at[wid, pl.ds(c * CB, CB), slice(None)], db, sems[b])
        return (h1, h2)

    pend = {0: start(0, 0)}
    for c in range(NCHUNK):
        b = c % 2
        if c + 1 < NCHUNK:
            pend[(c + 1) % 2] = start(c + 1, (c + 1) % 2)
        for h in pend.pop(b):
            h.wait()
        process_chunk(*bufs[b])


def _rsqrt16(x):
    xi = plsc.bitcast(x, jnp.int32)
    yi = jnp.full((16,), 0x5F3759DF, jnp.int32) - lax.shift_right_arithmetic(
        xi, jnp.full((16,), 1, jnp.int32))
    y = plsc.bitcast(yi, jnp.float32)
    for _ in range(4):
        y = y * (1.5 - 0.5 * x * y * y)
    return y


@functools.partial(
    pl.kernel,
    mesh=_mesh,
    compiler_params=_sc_params,
    out_type=[jax.ShapeDtypeStruct((2, NPAD), jnp.float32)] * 2,
    scratch_types=[
        pltpu.VMEM((128,), jnp.float32),
        pltpu.VMEM((SL,), jnp.float32),
        pltpu.VMEM((CB, 128), jnp.int32),
        pltpu.VMEM((CB, 128), jnp.int32),
        pltpu.VMEM((CB, 128), jnp.int32),
        pltpu.VMEM((CB, 128), jnp.int32),
        pltpu.VMEM_SHARED((NPAD,), jnp.float32),
        pltpu.VMEM_SHARED((NPAD,), jnp.float32),
        pltpu.SemaphoreType.DMA,
        pltpu.SemaphoreType.DMA,
    ],
)
def _sc_bincount(src_hbm, dst_hbm, outo_hbm, outi_hbm, ones, zbuf,
                 src0, dst0, src1, dst1, acc_o, acc_i, sem0, sem1):
    cid = lax.axis_index("c")
    sid = lax.axis_index("s")
    wid = cid * 16 + sid
    one16 = jnp.ones((16,), jnp.float32)
    z16 = jnp.zeros((16,), jnp.float32)

    for i in range(8):
        ones[pl.ds(i * 16, 16)] = one16

    def fill(i, carry):
        zbuf[pl.ds(i * 16, 16)] = z16
        return carry

    lax.fori_loop(0, SL // 16, fill, 0)
    pltpu.sync_copy(zbuf, acc_o.at[pl.ds(sid * SL, SL)])
    pltpu.sync_copy(zbuf, acc_i.at[pl.ds(sid * SL, SL)])
    plsc.subcore_barrier()

    def process(srcb, dstb):
        def row(b, carry2):
            pltpu.sync_copy(ones, acc_o.at[srcb.at[b]], add=True)
            pltpu.sync_copy(ones, acc_i.at[dstb.at[b]], add=True)
            return carry2

        lax.fori_loop(0, CB, row, 0)

    _edge_pipeline(src_hbm, dst_hbm, wid, [(src0, dst0), (src1, dst1)],
                   [sem0, sem1], process)
    plsc.subcore_barrier()

    @pl.when(sid == 0)
    def _out():
        pltpu.sync_copy(acc_o, outo_hbm.at[cid])
        pltpu.sync_copy(acc_i, outi_hbm.at[cid])


@functools.partial(
    pl.kernel,
    mesh=_mesh,
    compiler_params=_sc_params,
    out_type=jax.ShapeDtypeStruct((2, NPAD), jnp.float32),
    scratch_types=[
        pltpu.VMEM((NPAD,), jnp.float32),
        pltpu.VMEM((CB, 128), jnp.float32),
        pltpu.VMEM((SL,), jnp.float32),
        pltpu.VMEM((SL,), jnp.float32),
        pltpu.VMEM((CB, 128), jnp.int32),
        pltpu.VMEM((CB, 128), jnp.int32),
        pltpu.VMEM((CB, 128), jnp.int32),
        pltpu.VMEM((CB, 128), jnp.int32),
        pltpu.VMEM_SHARED((NPAD,), jnp.float32),
        pltpu.VMEM_SHARED((NPAD,), jnp.float32),
        pltpu.SemaphoreType.DMA,
        pltpu.SemaphoreType.DMA,
    ],
)
def _sc_pass1(src_hbm, dst_hbm, degi_hbm, out_hbm, tabv, vals, sb0, sb1,
              src0, dst0, src1, dst1, tab_sh, acc, sem0, sem1):
    cid = lax.axis_index("c")
    sid = lax.axis_index("s")
    wid = cid * 16 + sid
    base = sid * SL
    pltpu.sync_copy(degi_hbm.at[0, pl.ds(base, SL)], sb0)
    pltpu.sync_copy(degi_hbm.at[1, pl.ds(base, SL)], sb1)
    z16 = jnp.zeros((16,), jnp.float32)

    def prol(i, carry):
        d = jnp.maximum(sb0[pl.ds(i * 16, 16)] + sb1[pl.ds(i * 16, 16)], 1.0)
        sb0[pl.ds(i * 16, 16)] = _rsqrt16(d)
        sb1[pl.ds(i * 16, 16)] = z16
        return carry

    lax.fori_loop(0, SL // 16, prol, 0)
    pltpu.sync_copy(sb0, tab_sh.at[pl.ds(base, SL)])
    pltpu.sync_copy(sb1, acc.at[pl.ds(base, SL)])
    plsc.subcore_barrier()
    pltpu.sync_copy(tab_sh, tabv)

    def process(srcb, dstb):
        def grp(i, carry2):
            r = i // 2
            half = (i % 2) * 64
            for k in range(4):
                off = half + k * 16
                do = dstb[r, pl.ds(off, 16)]
                vals[r, pl.ds(off, 16)] = plsc.load_gather(tabv, [do])
            return carry2

        lax.fori_loop(0, CB * 2, grp, 0)

        def row(b, carry2):
            pltpu.sync_copy(vals.at[b], acc.at[srcb.at[b]], add=True)
            return carry2

        lax.fori_loop(0, CB, row, 0)

    _edge_pipeline(src_hbm, dst_hbm, wid, [(src0, dst0), (src1, dst1)],
                   [sem0, sem1], process)
    plsc.subcore_barrier()

    @pl.when(sid == 0)
    def _out():
        pltpu.sync_copy(acc, out_hbm.at[cid])


@functools.partial(
    pl.kernel,
    mesh=_mesh,
    compiler_params=_sc_params,
    out_type=[jax.ShapeDtypeStruct((2, NPAD), jnp.float32),
              jax.ShapeDtypeStruct((NPAD,), jnp.float32),
              jax.ShapeDtypeStruct((NPAD,), jnp.float32)],
    scratch_types=[
        pltpu.VMEM((NPAD,), jnp.float32),
        pltpu.VMEM((CB, 128), jnp.float32),
        pltpu.VMEM((SL,), jnp.float32),
        pltpu.VMEM((SL,), jnp.float32),
        pltpu.VMEM((SL,), jnp.float32),
        pltpu.VMEM((SL,), jnp.float32),
        pltpu.VMEM((SL,), jnp.float32),
        pltpu.VMEM((SL,), jnp.float32),
        pltpu.VMEM((CB, 128), jnp.int32),
        pltpu.VMEM((CB, 128), jnp.int32),
        pltpu.VMEM((CB, 128), jnp.int32),
        pltpu.VMEM((CB, 128), jnp.int32),
        pltpu.VMEM_SHARED((NPAD,), jnp.float32),
        pltpu.VMEM_SHARED((NPAD,), jnp.float32),
        pltpu.SemaphoreType.DMA,
        pltpu.SemaphoreType.DMA,
    ],
)
def _sc_pass2(src_hbm, dst_hbm, dego_hbm, degi_hbm, w_hbm, out_hbm, dd_hbm,
              v_hbm, tabv, vals, bi0, bi1, bo0, bo1, bw0, bw1,
              src0, dst0, src1, dst1, tab_sh, acc, sem0, sem1):
    cid = lax.axis_index("c")
    sid = lax.axis_index("s")
    wid = cid * 16 + sid
    base = sid * SL
    pltpu.sync_copy(degi_hbm.at[0, pl.ds(base, SL)], bi0)
    pltpu.sync_copy(degi_hbm.at[1, pl.ds(base, SL)], bi1)
    pltpu.sync_copy(dego_hbm.at[0, pl.ds(base, SL)], bo0)
    pltpu.sync_copy(dego_hbm.at[1, pl.ds(base, SL)], bo1)
    pltpu.sync_copy(w_hbm.at[0, pl.ds(base, SL)], bw0)
    pltpu.sync_copy(w_hbm.at[1, pl.ds(base, SL)], bw1)
    z16 = jnp.zeros((16,), jnp.float32)

    def prol(i, carry):
        sl = pl.ds(i * 16, 16)
        a = _rsqrt16(jnp.maximum(bi0[sl] + bi1[sl], 1.0))
        d = _rsqrt16(jnp.maximum(bo0[sl] + bo1[sl], 1.0))
        v = d * (bw0[sl] + bw1[sl])
        bo0[sl] = d
        bw0[sl] = v
        bi0[sl] = v * a
        bi1[sl] = z16
        return carry

    lax.fori_loop(0, SL // 16, prol, 0)

    @pl.when(cid == 0)
    def _ddv_out():
        pltpu.sync_copy(bo0, dd_hbm.at[pl.ds(base, SL)])
        pltpu.sync_copy(bw0, v_hbm.at[pl.ds(base, SL)])

    pltpu.sync_copy(bi0, tab_sh.at[pl.ds(base, SL)])
    pltpu.sync_copy(bi1, acc.at[pl.ds(base, SL)])
    plsc.subcore_barrier()
    pltpu.sync_copy(tab_sh, tabv)

    def process(srcb, dstb):
        def grp(i, carry2):
            r = i // 2
            half = (i % 2) * 64
            for k in range(4):
                off = half + k * 16
                do = dstb[r, pl.ds(off, 16)]
                vals[r, pl.ds(off, 16)] = plsc.load_gather(tabv, [do])
            return carry2

        lax.fori_loop(0, CB * 2, grp, 0)

        def row(b, carry2):
            pltpu.sync_copy(vals.at[b], acc.at[srcb.at[b]], add=True)
            return carry2

        lax.fori_loop(0, CB, row, 0)

    _edge_pipeline(src_hbm, dst_hbm, wid, [(src0, dst0), (src1, dst1)],
                   [sem0, sem1], process)
    plsc.subcore_barrier()

    @pl.when(sid == 0)
    def _out():
        pltpu.sync_copy(acc, out_hbm.at[cid])


def _tc_h0_body(x_ref, We, be, h0_ref):
    h0_ref[...] = jnp.maximum(
        jnp.dot(x_ref[...], We[...], preferred_element_type=jnp.float32,
                precision=lax.Precision.HIGHEST)
        + be[...], 0.0)


def _tc_main_body(h0_ref, t0, t1, ddr, vr, W1r, b1r, W2r, b2r,
                  Wcr, bcr, out_ref, zacc, svacc):
    i = pl.program_id(0)

    @pl.when(i == 0)
    def _init():
        zacc[...] = jnp.zeros_like(zacc)
        svacc[0] = 0.0

    idx = i * BN + lax.broadcasted_iota(jnp.int32, (1, BN), 1)
    m = (idx < N).astype(jnp.float32)
    u = ddr[0] * (t0[0] + t1[0]) * m
    svacc[0] += jnp.sum(vr[0] * m)
    zacc[...] += jnp.dot(u, h0_ref[...], preferred_element_type=jnp.float32,
                         precision=lax.Precision.HIGHEST)

    @pl.when(i == NBLK - 1)
    def _fin():
        z0 = zacc[...]
        sv = svacc[0]
        z1 = jnp.dot(z0, W1r[...], preferred_element_type=jnp.float32) \
            + sv * b1r[...]
        pooled = jnp.dot(z1, W2r[...], preferred_element_type=jnp.float32) \
            * (1.0 / N) + b2r[...]
        out_ref[...] = jnp.dot(pooled, Wcr[...],
                               preferred_element_type=jnp.float32) + bcr[...]


def kernel(x, edge_index, W_ext, b_ext, W1, b1, W2, b2, Wc, bc):
    src = edge_index[0].astype(jnp.int32)
    dst = edge_index[1].astype(jnp.int32)
    padi = jnp.full((EPAD - E,), N, dtype=jnp.int32)
    srcp = jnp.concatenate([src, padi]).reshape(NW, EP // 128, 128)
    dstp = jnp.concatenate([dst, padi]).reshape(NW, EP // 128, 128)

    x_pad = jnp.concatenate(
        [x, jnp.zeros((NPAD - N, x.shape[1]), x.dtype)], axis=0)
    full = lambda shp: pl.BlockSpec(shp, lambda i: tuple(0 for _ in shp))
    h0 = pl.pallas_call(
        _tc_h0_body,
        grid=(NBLK,),
        in_specs=[pl.BlockSpec((BN, 128), lambda i: (i, 0)),
                  full((128, 100)), full((1, 100))],
        out_specs=pl.BlockSpec((BN, 100), lambda i: (i, 0)),
        out_shape=jax.ShapeDtypeStruct((NPAD, 100), jnp.float32),
    )(x_pad, W_ext, b_ext.reshape(1, 100))

    dego, degi = _sc_bincount(srcp, dstp)
    w = _sc_pass1(srcp, dstp, degi)
    t, dd, v = _sc_pass2(srcp, dstp, dego, degi, w)

    row = pl.BlockSpec((1, 1, BN), lambda i: (i, 0, 0))
    out = pl.pallas_call(
        _tc_main_body,
        grid=(NBLK,),
        in_specs=[
            pl.BlockSpec((BN, 100), lambda i: (i, 0)),
            row, row,
            row, row,
            full((100, 100)), full((1, 100)),
            full((100, 200)), full((1, 200)),
            full((200, 55)), full((1, 55)),
        ],
        out_specs=full((1, 55)),
        out_shape=jax.ShapeDtypeStruct((1, 55), jnp.float32),
        scratch_shapes=[pltpu.VMEM((1, 100), jnp.float32),
                        pltpu.SMEM((1,), jnp.float32)],
    )(
        h0,
        t[0].reshape(NBLK, 1, BN), t[1].reshape(NBLK, 1, BN),
        dd.reshape(NBLK, 1, BN), v.reshape(NBLK, 1, BN),
        W1, b1.reshape(1, 100),
        W2, b2.reshape(1, 200),
        Wc, bc.reshape(1, 55),
    )
    return out


# speedup vs baseline: 36.3899x; 1.0105x over previous
import functools

import jax
import jax.numpy as jnp
from jax import lax
from jax.experimental import pallas as pl
from jax.experimental.pallas import tpu as pltpu
from jax.experimental.pallas import tpu_sc as plsc

N = 50000
NPAD = 51200
E = 800000
NW = 32
EP = 25600
EPAD = NW * EP
CH = 5120
CB = CH // 128
NCHUNK = EP // CH
SL = NPAD // 16
BN = 2048
NBLK = NPAD // BN
NR = 400
GB = 8
NG = NR // GB

_mesh = plsc.VectorSubcoreMesh(core_axis_name="c", subcore_axis_name="s")
_sc_params = pltpu.CompilerParams(needs_layout_passes=False)


def _edge_sweep(src_hbm, dst_hbm, wid, bufs, isems, fire, drain):
    def start(c, b):
        sb, db = bufs[b]
        h1 = pltpu.async_copy(
            src_hbm.at[wid, pl.ds(c * CB, CB), slice(None)], sb, isems[b])
        h2 = pltpu.async_copy(
            dst_hbm.at[wid, pl.ds(c * CB, CB), slice(None)], db, isems[b])
        return (h1, h2)

    pend = {0: start(0, 0)}
    for c in range(NCHUNK):
        b = c % 2
        if c + 1 < NCHUNK:
            if c >= 1:
                drain(1 - b)
            pend[1 - b] = start(c + 1, 1 - b)
        for h in pend.pop(b):
            h.wait()
        fire(b, *bufs[b])
    drain((NCHUNK - 2) % 2)
    drain((NCHUNK - 1) % 2)


def _rsqrt16(x):
    xi = plsc.bitcast(x, jnp.int32)
    yi = jnp.full((16,), 0x5F3759DF, jnp.int32) - lax.shift_right_arithmetic(
        xi, jnp.full((16,), 1, jnp.int32))
    y = plsc.bitcast(yi, jnp.float32)
    for _ in range(4):
        y = y * (1.5 - 0.5 * x * y * y)
    return y


@functools.partial(
    pl.kernel,
    mesh=_mesh,
    compiler_params=_sc_params,
    out_type=[jax.ShapeDtypeStruct((2, NPAD), jnp.float32)] * 2,
    scratch_types=[
        pltpu.VMEM((128,), jnp.float32),
        pltpu.VMEM((SL,), jnp.float32),
        pltpu.VMEM((CB, 128), jnp.int32),
        pltpu.VMEM((CB, 128), jnp.int32),
        pltpu.VMEM((CB, 128), jnp.int32),
        pltpu.VMEM((CB, 128), jnp.int32),
        pltpu.VMEM_SHARED((NPAD,), jnp.float32),
        pltpu.VMEM_SHARED((NPAD,), jnp.float32),
        pltpu.SemaphoreType.DMA,
        pltpu.SemaphoreType.DMA,
        pltpu.SemaphoreType.DMA,
        pltpu.SemaphoreType.DMA,
    ],
)
def _sc_bincount(src_hbm, dst_hbm, outo_hbm, outi_hbm, ones, zbuf,
                 src0, dst0, src1, dst1, acc_o, acc_i, sem0, sem1,
                 ssem0, ssem1):
    cid = lax.axis_index("c")
    sid = lax.axis_index("s")
    wid = cid * 16 + sid
    one16 = jnp.ones((16,), jnp.float32)
    z16 = jnp.zeros((16,), jnp.float32)

    for i in range(8):
        ones[pl.ds(i * 16, 16)] = one16

    def fill(i, carry):
        zbuf[pl.ds(i * 16, 16)] = z16
        return carry

    lax.fori_loop(0, SL // 16, fill, 0)
    pltpu.sync_copy(zbuf, acc_o.at[pl.ds(sid * SL, SL)])
    pltpu.sync_copy(zbuf, acc_i.at[pl.ds(sid * SL, SL)])
    plsc.subcore_barrier()

    ssems = (ssem0, ssem1)

    def fire(b, srcb, dstb):
        sem = ssems[b]

        def row(j, carry2):
            pltpu.async_copy(ones, acc_o.at[srcb.at[j]], sem, add=True)
            pltpu.async_copy(ones, acc_i.at[dstb.at[j]], sem, add=True)
            return carry2

        lax.fori_loop(0, CB, row, 0)

    def drain(b):
        sem = ssems[b]

        def row(j, carry2):
            pltpu.make_async_copy(outo_hbm.at[0, pl.ds(0, 128)], ones, sem).wait()
            pltpu.make_async_copy(outo_hbm.at[0, pl.ds(0, 128)], ones, sem).wait()
            return carry2

        lax.fori_loop(0, CB, row, 0)

    _edge_sweep(src_hbm, dst_hbm, wid, [(src0, dst0), (src1, dst1)],
                [sem0, sem1], fire, drain)
    plsc.subcore_barrier()

    @pl.when(sid == 0)
    def _out():
        pltpu.sync_copy(acc_o, outo_hbm.at[cid])
        pltpu.sync_copy(acc_i, outi_hbm.at[cid])


@functools.partial(
    pl.kernel,
    mesh=_mesh,
    compiler_params=_sc_params,
    out_type=jax.ShapeDtypeStruct((2, NPAD), jnp.float32),
    scratch_types=[
        pltpu.VMEM((NPAD,), jnp.float32),
        pltpu.VMEM((CB, 128), jnp.float32),
        pltpu.VMEM((CB, 128), jnp.float32),
        pltpu.VMEM((SL,), jnp.float32),
        pltpu.VMEM((SL,), jnp.float32),
        pltpu.VMEM((CB, 128), jnp.int32),
        pltpu.VMEM((CB, 128), jnp.int32),
        pltpu.VMEM((CB, 128), jnp.int32),
        pltpu.VMEM((CB, 128), jnp.int32),
        pltpu.VMEM_SHARED((NPAD,), jnp.float32),
        pltpu.VMEM_SHARED((NPAD,), jnp.float32),
        pltpu.SemaphoreType.DMA,
        pltpu.SemaphoreType.DMA,
        pltpu.SemaphoreType.DMA,
        pltpu.SemaphoreType.DMA,
    ],
)
def _sc_pass1(src_hbm, dst_hbm, degi_hbm, out_hbm, tabv, vals0, vals1,
              sb0, sb1, src0, dst0, src1, dst1, tab_sh, acc, sem0, sem1,
              ssem0, ssem1):
    cid = lax.axis_index("c")
    sid = lax.axis_index("s")
    wid = cid * 16 + sid
    base = sid * SL
    pltpu.sync_copy(degi_hbm.at[0, pl.ds(base, SL)], sb0)
    pltpu.sync_copy(degi_hbm.at[1, pl.ds(base, SL)], sb1)
    z16 = jnp.zeros((16,), jnp.float32)

    def prol(i, carry):
        d = jnp.maximum(sb0[pl.ds(i * 16, 16)] + sb1[pl.ds(i * 16, 16)], 1.0)
        sb0[pl.ds(i * 16, 16)] = _rsqrt16(d)
        sb1[pl.ds(i * 16, 16)] = z16
        return carry

    lax.fori_loop(0, SL // 16, prol, 0)
    pltpu.sync_copy(sb0, tab_sh.at[pl.ds(base, SL)])
    pltpu.sync_copy(sb1, acc.at[pl.ds(base, SL)])
    plsc.subcore_barrier()
    pltpu.sync_copy(tab_sh, tabv)

    ssems = (ssem0, ssem1)
    valsb2 = (vals0, vals1)

    def fire(b, srcb, dstb):
        valsb = valsb2[b]
        sem = ssems[b]

        def grp(i, carry2):
            r = i // 2
            half = (i % 2) * 64
            for k in range(4):
                off = half + k * 16
                do = dstb[r, pl.ds(off, 16)]
                valsb[r, pl.ds(off, 16)] = plsc.load_gather(tabv, [do])
            return carry2

        lax.fori_loop(0, CB * 2, grp, 0)

        def row(j, carry2):
            pltpu.async_copy(valsb.at[j], acc.at[srcb.at[j]], sem, add=True)
            return carry2

        lax.fori_loop(0, CB, row, 0)

    def drain(b):
        sem = ssems[b]

        def row(j, carry2):
            pltpu.make_async_copy(degi_hbm.at[0, pl.ds(0, 128)], vals0.at[0],
                                  sem).wait()
            return carry2

        lax.fori_loop(0, CB, row, 0)

    _edge_sweep(src_hbm, dst_hbm, wid, [(src0, dst0), (src1, dst1)],
                [sem0, sem1], fire, drain)
    plsc.subcore_barrier()

    @pl.when(sid == 0)
    def _out():
        pltpu.sync_copy(acc, out_hbm.at[cid])


@functools.partial(
    pl.kernel,
    mesh=_mesh,
    compiler_params=_sc_params,
    out_type=[jax.ShapeDtypeStruct((2, NPAD), jnp.float32),
              jax.ShapeDtypeStruct((NPAD,), jnp.float32),
              jax.ShapeDtypeStruct((NPAD,), jnp.float32)],
    scratch_types=[
        pltpu.VMEM((NPAD,), jnp.float32),
        pltpu.VMEM((CB, 128), jnp.float32),
        pltpu.VMEM((CB, 128), jnp.float32),
        pltpu.VMEM((SL,), jnp.float32),
        pltpu.VMEM((SL,), jnp.float32),
        pltpu.VMEM((SL,), jnp.float32),
        pltpu.VMEM((SL,), jnp.float32),
        pltpu.VMEM((SL,), jnp.float32),
        pltpu.VMEM((SL,), jnp.float32),
        pltpu.VMEM((CB, 128), jnp.int32),
        pltpu.VMEM((CB, 128), jnp.int32),
        pltpu.VMEM((CB, 128), jnp.int32),
        pltpu.VMEM((CB, 128), jnp.int32),
        pltpu.VMEM_SHARED((NPAD,), jnp.float32),
        pltpu.VMEM_SHARED((NPAD,), jnp.float32),
        pltpu.SemaphoreType.DMA,
        pltpu.SemaphoreType.DMA,
        pltpu.SemaphoreType.DMA,
        pltpu.SemaphoreType.DMA,
    ],
)
def _sc_pass2(src_hbm, dst_hbm, dego_hbm, degi_hbm, w_hbm, out_hbm, dd_hbm,
              v_hbm, tabv, vals0, vals1, bi0, bi1, bo0, bo1, bw0, bw1,
              src0, dst0, src1, dst1, tab_sh, acc, sem0, sem1,
              ssem0, ssem1):
    cid = lax.axis_index("c")
    sid = lax.axis_index("s")
    wid = cid * 16 + sid
    base = sid * SL
    pltpu.sync_copy(degi_hbm.at[0, pl.ds(base, SL)], bi0)
    pltpu.sync_copy(degi_hbm.at[1, pl.ds(base, SL)], bi1)
    pltpu.sync_copy(dego_hbm.at[0, pl.ds(base, SL)], bo0)
    pltpu.sync_copy(dego_hbm.at[1, pl.ds(base, SL)], bo1)
    pltpu.sync_copy(w_hbm.at[0, pl.ds(base, SL)], bw0)
    pltpu.sync_copy(w_hbm.at[1, pl.ds(base, SL)], bw1)
    z16 = jnp.zeros((16,), jnp.float32)

    def prol(i, carry):
        sl = pl.ds(i * 16, 16)
        a = _rsqrt16(jnp.maximum(bi0[sl] + bi1[sl], 1.0))
        d = _rsqrt16(jnp.maximum(bo0[sl] + bo1[sl], 1.0))
        v = d * (bw0[sl] + bw1[sl])
        bo0[sl] = d
        bw0[sl] = v
        bi0[sl] = v * a
        bi1[sl] = z16
        return carry

    lax.fori_loop(0, SL // 16, prol, 0)

    @pl.when(cid == 0)
    def _ddv_out():
        pltpu.sync_copy(bo0, dd_hbm.at[pl.ds(base, SL)])
        pltpu.sync_copy(bw0, v_hbm.at[pl.ds(base, SL)])

    pltpu.sync_copy(bi0, tab_sh.at[pl.ds(base, SL)])
    pltpu.sync_copy(bi1, acc.at[pl.ds(base, SL)])
    plsc.subcore_barrier()
    pltpu.sync_copy(tab_sh, tabv)

    ssems = (ssem0, ssem1)
    valsb2 = (vals0, vals1)

    def fire(b, srcb, dstb):
        valsb = valsb2[b]
        sem = ssems[b]

        def grp(i, carry2):
            r = i // 2
            half = (i % 2) * 64
            for k in range(4):
                off = half + k * 16
                do = dstb[r, pl.ds(off, 16)]
                valsb[r, pl.ds(off, 16)] = plsc.load_gather(tabv, [do])
            return carry2

        lax.fori_loop(0, CB * 2, grp, 0)

        def row(j, carry2):
            pltpu.async_copy(valsb.at[j], acc.at[srcb.at[j]], sem, add=True)
            return carry2

        lax.fori_loop(0, CB, row, 0)

    def drain(b):
        sem = ssems[b]

        def row(j, carry2):
            pltpu.make_async_copy(degi_hbm.at[0, pl.ds(0, 128)], vals0.at[0],
                                  sem).wait()
            return carry2

        lax.fori_loop(0, CB, row, 0)

    _edge_sweep(src_hbm, dst_hbm, wid, [(src0, dst0), (src1, dst1)],
                [sem0, sem1], fire, drain)
    plsc.subcore_barrier()

    @pl.when(sid == 0)
    def _out():
        pltpu.sync_copy(acc, out_hbm.at[cid])


def _tc_h0_body(x_ref, We, be, h0_ref):
    h0_ref[...] = jnp.maximum(
        jnp.dot(x_ref[...], We[...], preferred_element_type=jnp.float32,
                precision=lax.Precision.HIGHEST)
        + be[...], 0.0)


def _tc_main_body(h0_ref, t0, t1, ddr, vr, W1r, b1r, W2r, b2r,
                  Wcr, bcr, out_ref, zacc, svacc):
    i = pl.program_id(0)

    @pl.when(i == 0)
    def _init():
        zacc[...] = jnp.zeros_like(zacc)
        svacc[0] = 0.0

    idx = i * BN + lax.broadcasted_iota(jnp.int32, (1, BN), 1)
    m = (idx < N).astype(jnp.float32)
    u = ddr[0] * (t0[0] + t1[0]) * m
    svacc[0] += jnp.sum(vr[0] * m)
    zacc[...] += jnp.dot(u, h0_ref[...], preferred_element_type=jnp.float32,
                         precision=lax.Precision.HIGHEST)

    @pl.when(i == NBLK - 1)
    def _fin():
        z0 = zacc[...]
        sv = svacc[0]
        z1 = jnp.dot(z0, W1r[...], preferred_element_type=jnp.float32) \
            + sv * b1r[...]
        pooled = jnp.dot(z1, W2r[...], preferred_element_type=jnp.float32) \
            * (1.0 / N) + b2r[...]
        out_ref[...] = jnp.dot(pooled, Wcr[...],
                               preferred_element_type=jnp.float32) + bcr[...]


def kernel(x, edge_index, W_ext, b_ext, W1, b1, W2, b2, Wc, bc):
    src = edge_index[0].astype(jnp.int32)
    dst = edge_index[1].astype(jnp.int32)
    padi = jnp.full((EPAD - E,), N, dtype=jnp.int32)
    srcp = jnp.concatenate([src, padi]).reshape(NW, EP // 128, 128)
    dstp = jnp.concatenate([dst, padi]).reshape(NW, EP // 128, 128)

    x_pad = jnp.concatenate(
        [x, jnp.zeros((NPAD - N, x.shape[1]), x.dtype)], axis=0)
    full = lambda shp: pl.BlockSpec(shp, lambda i: tuple(0 for _ in shp))
    h0 = pl.pallas_call(
        _tc_h0_body,
        grid=(NBLK,),
        in_specs=[pl.BlockSpec((BN, 128), lambda i: (i, 0)),
                  full((128, 100)), full((1, 100))],
        out_specs=pl.BlockSpec((BN, 100), lambda i: (i, 0)),
        out_shape=jax.ShapeDtypeStruct((NPAD, 100), jnp.float32),
    )(x_pad, W_ext, b_ext.reshape(1, 100))

    dego, degi = _sc_bincount(srcp, dstp)
    w = _sc_pass1(srcp, dstp, degi)
    t, dd, v = _sc_pass2(srcp, dstp, dego, degi, w)

    row = pl.BlockSpec((1, 1, BN), lambda i: (i, 0, 0))
    out = pl.pallas_call(
        _tc_main_body,
        grid=(NBLK,),
        in_specs=[
            pl.BlockSpec((BN, 100), lambda i: (i, 0)),
            row, row,
            row, row,
            full((100, 100)), full((1, 100)),
            full((100, 200)), full((1, 200)),
            full((200, 55)), full((1, 55)),
        ],
        out_specs=full((1, 55)),
        out_shape=jax.ShapeDtypeStruct((1, 55), jnp.float32),
        scratch_shapes=[pltpu.VMEM((1, 100), jnp.float32),
                        pltpu.SMEM((1,), jnp.float32)],
    )(
        h0,
        t[0].reshape(NBLK, 1, BN), t[1].reshape(NBLK, 1, BN),
        dd.reshape(NBLK, 1, BN), v.reshape(NBLK, 1, BN),
        W1, b1.reshape(1, 100),
        W2, b2.reshape(1, 200),
        Wc, bc.reshape(1, 55),
    )
    return out
